# baseline (speedup 1.0000x reference)
import functools

import jax
import jax.numpy as jnp
from jax import lax
from jax.experimental import pallas as pl
from jax.experimental.pallas import tpu as pltpu
from jax.experimental.pallas import tpu_sc as plsc

VOCAB = 1000
D = 1000
DPAD = 1024

NC = 2
NS = 16
NW = NC * NS


def _make_gather(B: int, S: int):
    assert B % NW == 0
    b_per_w = B // NW

    mesh = plsc.VectorSubcoreMesh(core_axis_name="c", subcore_axis_name="s")

    @functools.partial(
        pl.kernel,
        mesh=mesh,
        out_type=jax.ShapeDtypeStruct((B, S, D), jnp.float32),
        scratch_types=[
            pltpu.VMEM((S,), jnp.int32),
            pltpu.VMEM((S, DPAD), jnp.float32),
            pltpu.SemaphoreType.DMA,
        ],
        compiler_params=pltpu.CompilerParams(use_tc_tiling_on_sc=False),
    )
    def gather(idx_hbm, table_hbm, out_hbm, idx_v, rows_v, sem):
        wid = lax.axis_index("s") * NC + lax.axis_index("c")
        base = wid * b_per_w

        def body(j, carry):
            bi = base + j
            pltpu.sync_copy(idx_hbm.at[bi], idx_v)
            pltpu.async_copy(table_hbm.at[idx_v], rows_v, sem).wait()
            pltpu.sync_copy(rows_v.at[:, pl.ds(0, D)], out_hbm.at[bi])
            return carry

        lax.fori_loop(0, b_per_w, body, 0)

    return gather


def kernel(idx, token_table):
    b, s = idx.shape
    table_pad = jnp.pad(token_table, ((0, 0), (0, DPAD - D)))
    return _make_gather(b, s)(idx.astype(jnp.int32), table_pad)


# speedup vs baseline: 1.0477x; 1.0477x over previous
import functools

import jax
import jax.numpy as jnp
from jax import lax
from jax.experimental import pallas as pl
from jax.experimental.pallas import tpu as pltpu
from jax.experimental.pallas import tpu_sc as plsc

VOCAB = 1000
D = 1000
DPAD = 1024

NC = 2
NS = 16
NW = NC * NS


def _make_gather(B: int, S: int):
    assert B % (2 * NW) == 0
    b_per_w = B // NW
    n_pairs = b_per_w // 2

    mesh = plsc.VectorSubcoreMesh(core_axis_name="c", subcore_axis_name="s")

    @functools.partial(
        pl.kernel,
        mesh=mesh,
        out_type=jax.ShapeDtypeStruct((B, S, D), jnp.float32),
        scratch_types=[
            pltpu.VMEM((b_per_w, S), jnp.int32),
            pltpu.VMEM((S, DPAD), jnp.float32),
            pltpu.VMEM((S, DPAD), jnp.float32),
            pltpu.SemaphoreType.DMA,
            pltpu.SemaphoreType.DMA,
            pltpu.SemaphoreType.DMA,
            pltpu.SemaphoreType.DMA,
        ],
        compiler_params=pltpu.CompilerParams(use_tc_tiling_on_sc=False),
    )
    def gather(idx_hbm, table_hbm, out_hbm, idx_v, buf0, buf1, g0, g1, o0, o1):
        wid = lax.axis_index("s") * NC + lax.axis_index("c")
        base = wid * b_per_w

        pltpu.sync_copy(idx_hbm.at[pl.ds(base, b_per_w)], idx_v)

        def gstart(buf, gsem, j):
            pltpu.async_copy(table_hbm.at[idx_v.at[j]], buf, gsem)

        def gwait(buf, gsem, j):
            pltpu.make_async_copy(table_hbm.at[idx_v.at[j]], buf, gsem).wait()

        def ostart(buf, osem, j):
            pltpu.async_copy(buf.at[:, pl.ds(0, D)], out_hbm.at[base + j], osem)

        def owait(buf, osem, j):
            pltpu.make_async_copy(
                buf.at[:, pl.ds(0, D)], out_hbm.at[base + j], osem).wait()

        gstart(buf0, g0, 0)
        gstart(buf1, g1, 1)

        def body(jj, carry):
            j0 = 2 * jj
            gwait(buf0, g0, j0)
            ostart(buf0, o0, j0)
            gwait(buf1, g1, j0 + 1)
            ostart(buf1, o1, j0 + 1)
            owait(buf0, o0, j0)
            gstart(buf0, g0, j0 + 2)
            owait(buf1, o1, j0 + 1)
            gstart(buf1, g1, j0 + 3)
            return carry

        lax.fori_loop(0, n_pairs - 1, body, 0)

        j0 = b_per_w - 2
        gwait(buf0, g0, j0)
        ostart(buf0, o0, j0)
        gwait(buf1, g1, j0 + 1)
        ostart(buf1, o1, j0 + 1)
        owait(buf0, o0, j0)
        owait(buf1, o1, j0 + 1)

    return gather


def kernel(idx, token_table):
    b, s = idx.shape
    table_pad = jnp.pad(token_table, ((0, 0), (0, DPAD - D)))
    return _make_gather(b, s)(idx.astype(jnp.int32), table_pad)
